# baseline (speedup 1.0000x reference)
import functools

import jax
import jax.numpy as jnp
from jax import lax
from jax.experimental import pallas as pl
from jax.experimental.pallas import tpu as pltpu
from jax.experimental.pallas import tpu_sc as plsc

_N = 10000
_E = 320000
_D = 128
_R = 200
_RP = 256

_K = 5
_EC = _E // _K

_NC = 2
_NS = 16
_NW = _NC * _NS
_PER_W = _EC // _NW
_C = 128
_NFULL = _PER_W // _C
_TAIL = _PER_W - _NFULL * _C
_PAIRS = _NFULL // 2

_STAGE = 632
_STAGE_LAST = _N - 15 * _STAGE


def _sc_gather_body(node_hbm, idx_hbm, out_hbm, table_sp, idx_v, rows0, rows1,
                    g0, g1):
    sid = lax.axis_index("s")
    wid = sid * _NC + lax.axis_index("c")
    base = pl.multiple_of(wid * _PER_W, 8)

    @pl.when(sid < _NS - 1)
    def _():
        soff = pl.multiple_of(sid * _STAGE, 8)
        pltpu.sync_copy(
            node_hbm.at[pl.ds(soff, _STAGE)],
            table_sp.at[pl.ds(soff, _STAGE)],
        )

    @pl.when(sid == _NS - 1)
    def _():
        pltpu.sync_copy(
            node_hbm.at[pl.ds(15 * _STAGE, _STAGE_LAST)],
            table_sp.at[pl.ds(15 * _STAGE, _STAGE_LAST)],
        )

    pltpu.sync_copy(idx_hbm.at[pl.ds(base, _PER_W)], idx_v)
    plsc.subcore_barrier()

    bufs = (rows0, rows1)
    sems = (g0, g1)

    def start(j, b):
        off = pl.multiple_of(j * _C, 8)
        return pltpu.async_copy(
            table_sp.at[idx_v.at[pl.ds(off, _C)]], bufs[b], sems[b]
        )

    def drain(j, b):
        pltpu.make_async_copy(
            table_sp.at[idx_v.at[pl.ds(0, _C)]], bufs[b], sems[b]
        ).wait()
        off = pl.multiple_of(j * _C, 8)
        pltpu.sync_copy(bufs[b], out_hbm.at[pl.ds(base + off, _C)])

    start(0, 0)

    def pair(m, carry):
        j0 = 2 * m
        start(j0 + 1, 1)
        drain(j0, 0)

        @pl.when(m + 1 < _PAIRS)
        def _():
            start(j0 + 2, 0)

        drain(j0 + 1, 1)
        return carry

    lax.fori_loop(0, _PAIRS, pair, 0)
    if _NFULL % 2:
        start(_NFULL - 1, 0)
        drain(_NFULL - 1, 0)
    if _TAIL:
        toff = pl.multiple_of(_NFULL * _C, 8)
        pltpu.async_copy(
            table_sp.at[idx_v.at[pl.ds(toff, _TAIL)]],
            rows1.at[pl.ds(0, _TAIL)],
            g1,
        ).wait()
        pltpu.sync_copy(
            rows1.at[pl.ds(0, _TAIL)], out_hbm.at[pl.ds(base + toff, _TAIL)]
        )


def _sc_gather(node_feat, idx_chunk):
    mesh = plsc.VectorSubcoreMesh(core_axis_name="c", subcore_axis_name="s")
    fn = functools.partial(
        pl.kernel,
        mesh=mesh,
        out_type=jax.ShapeDtypeStruct((_EC, _D), jnp.float32),
        scratch_types=[
            pltpu.VMEM_SHARED((_N, _D), jnp.float32),
            pltpu.VMEM((_PER_W,), jnp.int32),
            pltpu.VMEM((_C, _D), jnp.float32),
            pltpu.VMEM((_C, _D), jnp.float32),
            pltpu.SemaphoreType.DMA,
            pltpu.SemaphoreType.DMA,
        ],
    )(_sc_gather_body)
    return fn(node_feat, idx_chunk)


_B = 12800
_G = _B // 128
_CB = _EC // _B
_ER = _E // 128


def _tc_math(et, src, ef, emb, wur, wc, bu, br):
    iota_r = lax.broadcasted_iota(jnp.int32, (_RP, 128), 0)
    gs = []
    for t in range(_G):
        oht = (iota_r == et[0, t:t + 1, :]).astype(jnp.float32)
        gs.append(lax.dot_general(
            oht, emb, (((0,), (0,)), ((), ())),
            preferred_element_type=jnp.float32))
    g = jnp.concatenate(gs, axis=0)
    pur = jnp.dot(src, wur, preferred_element_type=jnp.float32)
    u = jax.nn.sigmoid(g[:, :_D] * ef + pur[:, :_D] + bu)
    r = jax.nn.sigmoid(g[:, _D:2 * _D] * ef + pur[:, _D:] + br)
    c = jnp.tanh(
        g[:, 2 * _D:] * ef
        + jnp.dot(r * src, wc, preferred_element_type=jnp.float32)
    )
    return u * c + (1.0 - u) * src


def _tc_body(et_ref, src_ref, ef_ref, emb_ref, wur_ref, wc_ref, bu_ref, br_ref,
             out_ref):
    out_ref[...] = _tc_math(
        et_ref[...], src_ref[...], ef_ref[...], emb_ref[...], wur_ref[...],
        wc_ref[...], bu_ref[...], br_ref[...])


def _tc_body_alias(et_ref, src_ref, ef_ref, emb_ref, wur_ref, wc_ref, bu_ref,
                   br_ref, buf_ref, out_ref):
    out_ref[...] = _tc_math(
        et_ref[...], src_ref[...], ef_ref[...], emb_ref[...], wur_ref[...],
        wc_ref[...], bu_ref[...], br_ref[...])


def _tc_chunk(et2, src_k, edge_feat, emb_all, wur, wc, bu, br, buf, k):
    base_specs = [
        pl.BlockSpec((1, _G, 128), lambda i, k=k: (k * _CB + i, 0, 0)),
        pl.BlockSpec((_B, _D), lambda i: (i, 0)),
        pl.BlockSpec((_B, _D), lambda i, k=k: (k * _CB + i, 0)),
        pl.BlockSpec((_RP, 3 * _D), lambda i: (0, 0)),
        pl.BlockSpec((_D, 2 * _D), lambda i: (0, 0)),
        pl.BlockSpec((_D, _D), lambda i: (0, 0)),
        pl.BlockSpec((1, _D), lambda i: (0, 0)),
        pl.BlockSpec((1, _D), lambda i: (0, 0)),
    ]
    out_spec = pl.BlockSpec((_B, _D), lambda i, k=k: (k * _CB + i, 0))
    out_shape = jax.ShapeDtypeStruct((_E, _D), jnp.float32)
    if buf is None:
        return pl.pallas_call(
            _tc_body,
            grid=(_CB,),
            in_specs=base_specs,
            out_specs=out_spec,
            out_shape=out_shape,
        )(et2, src_k, edge_feat, emb_all, wur, wc, bu, br)
    return pl.pallas_call(
        _tc_body_alias,
        grid=(_CB,),
        in_specs=base_specs + [pl.BlockSpec((8, _D), lambda i: (0, 0))],
        out_specs=out_spec,
        out_shape=out_shape,
        input_output_aliases={8: 0},
    )(et2, src_k, edge_feat, emb_all, wur, wc, bu, br, buf)


def kernel(node_feat, edge_feat, src_idx, edge_type, emb_update, emb_reset,
           emb_candidate, W_update, b_update, W_reset, b_reset, W_candidate):
    src_idx = src_idx.astype(jnp.int32)
    edge_type = edge_type.astype(jnp.int32)
    emb_all = jnp.pad(
        jnp.concatenate([emb_update, emb_reset, emb_candidate], axis=1),
        ((0, _RP - _R), (0, 0)),
    )
    wur = jnp.concatenate([W_update.T, W_reset.T], axis=1)
    et3 = edge_type.reshape(_E // (_G * 128), _G, 128)
    bu = b_update.reshape(1, _D)
    br = b_reset.reshape(1, _D)
    wc = W_candidate.T

    idx_chunks = src_idx.reshape(_K, _EC)
    src_chunks = [_sc_gather(node_feat, idx_chunks[k]) for k in range(_K)]
    buf = None
    for k in range(_K):
        buf = _tc_chunk(et3, src_chunks[k], edge_feat, emb_all, wur, wc, bu,
                        br, buf, k)
    return buf


# speedup vs baseline: 1.0394x; 1.0394x over previous
import functools

import jax
import jax.numpy as jnp
from jax import lax
from jax.experimental import pallas as pl
from jax.experimental.pallas import tpu as pltpu
from jax.experimental.pallas import tpu_sc as plsc

_N = 10000
_E = 320000
_D = 128
_R = 200
_RP = 256

_K = 5
_EC = _E // _K

_NC = 2
_NS = 16
_NW = _NC * _NS
_PER_W = _EC // _NW
_C = 128
_NFULL = _PER_W // _C
_TAIL = _PER_W - _NFULL * _C
_PAIRS = _NFULL // 2

_STAGE = 632
_STAGE_LAST = _N - 15 * _STAGE


def _sc_gather_body(node_hbm, idx_hbm, out_hbm, table_sp, idx_v, rows0, rows1,
                    g0, g1):
    sid = lax.axis_index("s")
    wid = sid * _NC + lax.axis_index("c")
    base = pl.multiple_of(wid * _PER_W, 8)

    @pl.when(sid < _NS - 1)
    def _():
        soff = pl.multiple_of(sid * _STAGE, 8)
        pltpu.sync_copy(
            node_hbm.at[pl.ds(soff, _STAGE)],
            table_sp.at[pl.ds(soff, _STAGE)],
        )

    @pl.when(sid == _NS - 1)
    def _():
        pltpu.sync_copy(
            node_hbm.at[pl.ds(15 * _STAGE, _STAGE_LAST)],
            table_sp.at[pl.ds(15 * _STAGE, _STAGE_LAST)],
        )

    pltpu.sync_copy(idx_hbm.at[pl.ds(base, _PER_W)], idx_v)
    plsc.subcore_barrier()

    bufs = (rows0, rows1)
    sems = (g0, g1)

    def start(j, b):
        off = pl.multiple_of(j * _C, 8)
        return pltpu.async_copy(
            table_sp.at[idx_v.at[pl.ds(off, _C)]], bufs[b], sems[b]
        )

    def drain(j, b):
        pltpu.make_async_copy(
            table_sp.at[idx_v.at[pl.ds(0, _C)]], bufs[b], sems[b]
        ).wait()
        off = pl.multiple_of(j * _C, 8)
        pltpu.sync_copy(bufs[b], out_hbm.at[pl.ds(base + off, _C)])

    start(0, 0)

    def pair(m, carry):
        j0 = 2 * m
        start(j0 + 1, 1)
        drain(j0, 0)

        @pl.when(m + 1 < _PAIRS)
        def _():
            start(j0 + 2, 0)

        drain(j0 + 1, 1)
        return carry

    lax.fori_loop(0, _PAIRS, pair, 0)
    if _NFULL % 2:
        start(_NFULL - 1, 0)
        drain(_NFULL - 1, 0)
    if _TAIL:
        toff = pl.multiple_of(_NFULL * _C, 8)
        pltpu.async_copy(
            table_sp.at[idx_v.at[pl.ds(toff, _TAIL)]],
            rows1.at[pl.ds(0, _TAIL)],
            g1,
        ).wait()
        pltpu.sync_copy(
            rows1.at[pl.ds(0, _TAIL)], out_hbm.at[pl.ds(base + toff, _TAIL)]
        )


def _sc_gather(node_feat, idx_chunk):
    mesh = plsc.VectorSubcoreMesh(core_axis_name="c", subcore_axis_name="s")
    fn = functools.partial(
        pl.kernel,
        mesh=mesh,
        out_type=jax.ShapeDtypeStruct((_EC, _D), jnp.float32),
        scratch_types=[
            pltpu.VMEM_SHARED((_N, _D), jnp.float32),
            pltpu.VMEM((_PER_W,), jnp.int32),
            pltpu.VMEM((_C, _D), jnp.float32),
            pltpu.VMEM((_C, _D), jnp.float32),
            pltpu.SemaphoreType.DMA,
            pltpu.SemaphoreType.DMA,
        ],
    )(_sc_gather_body)
    return fn(node_feat, idx_chunk)


_B = 6400
_G = _B // 128
_CB = _EC // _B
_ER = _E // 128


def _tc_math(et, src, ef, emb, wur, wc, bu, br):
    iota_r = lax.broadcasted_iota(jnp.int32, (_RP, 128), 0)
    gs = []
    for t in range(_G):
        oht = (iota_r == et[0, t:t + 1, :]).astype(jnp.float32)
        gs.append(lax.dot_general(
            oht, emb, (((0,), (0,)), ((), ())),
            preferred_element_type=jnp.float32))
    g = jnp.concatenate(gs, axis=0)
    pur = jnp.dot(src, wur, preferred_element_type=jnp.float32)
    u = jax.nn.sigmoid(g[:, :_D] * ef + pur[:, :_D] + bu)
    r = jax.nn.sigmoid(g[:, _D:2 * _D] * ef + pur[:, _D:] + br)
    c = jnp.tanh(
        g[:, 2 * _D:] * ef
        + jnp.dot(r * src, wc, preferred_element_type=jnp.float32)
    )
    return u * c + (1.0 - u) * src


def _tc_body(et_ref, src_ref, ef_ref, emb_ref, wur_ref, wc_ref, bu_ref, br_ref,
             out_ref):
    out_ref[...] = _tc_math(
        et_ref[...], src_ref[...], ef_ref[...], emb_ref[...], wur_ref[...],
        wc_ref[...], bu_ref[...], br_ref[...])


def _tc_body_alias(et_ref, src_ref, ef_ref, emb_ref, wur_ref, wc_ref, bu_ref,
                   br_ref, buf_ref, out_ref):
    out_ref[...] = _tc_math(
        et_ref[...], src_ref[...], ef_ref[...], emb_ref[...], wur_ref[...],
        wc_ref[...], bu_ref[...], br_ref[...])


def _tc_chunk(et2, src_k, edge_feat, emb_all, wur, wc, bu, br, buf, k):
    base_specs = [
        pl.BlockSpec((1, _G, 128), lambda i, k=k: (k * _CB + i, 0, 0)),
        pl.BlockSpec((_B, _D), lambda i: (i, 0)),
        pl.BlockSpec((_B, _D), lambda i, k=k: (k * _CB + i, 0)),
        pl.BlockSpec((_RP, 3 * _D), lambda i: (0, 0)),
        pl.BlockSpec((_D, 2 * _D), lambda i: (0, 0)),
        pl.BlockSpec((_D, _D), lambda i: (0, 0)),
        pl.BlockSpec((1, _D), lambda i: (0, 0)),
        pl.BlockSpec((1, _D), lambda i: (0, 0)),
    ]
    out_spec = pl.BlockSpec((_B, _D), lambda i, k=k: (k * _CB + i, 0))
    out_shape = jax.ShapeDtypeStruct((_E, _D), jnp.float32)
    if buf is None:
        return pl.pallas_call(
            _tc_body,
            grid=(_CB,),
            in_specs=base_specs,
            out_specs=out_spec,
            out_shape=out_shape,
        )(et2, src_k, edge_feat, emb_all, wur, wc, bu, br)
    return pl.pallas_call(
        _tc_body_alias,
        grid=(_CB,),
        in_specs=base_specs + [pl.BlockSpec((8, _D), lambda i: (0, 0))],
        out_specs=out_spec,
        out_shape=out_shape,
        input_output_aliases={8: 0},
    )(et2, src_k, edge_feat, emb_all, wur, wc, bu, br, buf)


def kernel(node_feat, edge_feat, src_idx, edge_type, emb_update, emb_reset,
           emb_candidate, W_update, b_update, W_reset, b_reset, W_candidate):
    src_idx = src_idx.astype(jnp.int32)
    edge_type = edge_type.astype(jnp.int32)
    emb_all = jnp.pad(
        jnp.concatenate([emb_update, emb_reset, emb_candidate], axis=1),
        ((0, _RP - _R), (0, 0)),
    )
    wur = jnp.concatenate([W_update.T, W_reset.T], axis=1)
    et3 = edge_type.reshape(_E // (_G * 128), _G, 128)
    bu = b_update.reshape(1, _D)
    br = b_reset.reshape(1, _D)
    wc = W_candidate.T

    idx_chunks = src_idx.reshape(_K, _EC)
    src_chunks = [_sc_gather(node_feat, idx_chunks[k]) for k in range(_K)]
    buf = None
    for k in range(_K):
        buf = _tc_chunk(et3, src_chunks[k], edge_feat, emb_all, wur, wc, bu,
                        br, buf, k)
    return buf


# speedup vs baseline: 1.0702x; 1.0296x over previous
import functools

import jax
import jax.numpy as jnp
from jax import lax
from jax.experimental import pallas as pl
from jax.experimental.pallas import tpu as pltpu
from jax.experimental.pallas import tpu_sc as plsc

_N = 10000
_E = 320000
_D = 128
_R = 200
_RP = 256

_K = 5
_CHUNKS = (32000, 57600, 76800, 76800, 76800)
_OFFS = (0, 32000, 89600, 166400, 243200)

_NC = 2
_NS = 16
_NW = _NC * _NS
_C = 128

_STAGE = 632
_STAGE_LAST = _N - 15 * _STAGE


def _make_sc_body(per_w):
  nfull = per_w // _C
  tail = per_w - nfull * _C
  pairs = nfull // 2

  def _sc_gather_body(node_hbm, idx_hbm, out_hbm, table_sp, idx_v, rows0,
                      rows1, g0, g1):
    sid = lax.axis_index("s")
    wid = sid * _NC + lax.axis_index("c")
    base = pl.multiple_of(wid * per_w, 8)

    @pl.when(sid < _NS - 1)
    def _():
        soff = pl.multiple_of(sid * _STAGE, 8)
        pltpu.sync_copy(
            node_hbm.at[pl.ds(soff, _STAGE)],
            table_sp.at[pl.ds(soff, _STAGE)],
        )

    @pl.when(sid == _NS - 1)
    def _():
        pltpu.sync_copy(
            node_hbm.at[pl.ds(15 * _STAGE, _STAGE_LAST)],
            table_sp.at[pl.ds(15 * _STAGE, _STAGE_LAST)],
        )

    pltpu.sync_copy(idx_hbm.at[pl.ds(base, per_w)], idx_v)
    plsc.subcore_barrier()

    bufs = (rows0, rows1)
    sems = (g0, g1)

    def start(j, b):
        off = pl.multiple_of(j * _C, 8)
        return pltpu.async_copy(
            table_sp.at[idx_v.at[pl.ds(off, _C)]], bufs[b], sems[b]
        )

    def drain(j, b):
        pltpu.make_async_copy(
            table_sp.at[idx_v.at[pl.ds(0, _C)]], bufs[b], sems[b]
        ).wait()
        off = pl.multiple_of(j * _C, 8)
        pltpu.sync_copy(bufs[b], out_hbm.at[pl.ds(base + off, _C)])

    start(0, 0)

    def pair(m, carry):
        j0 = 2 * m
        start(j0 + 1, 1)
        drain(j0, 0)

        @pl.when(m + 1 < pairs)
        def _():
            start(j0 + 2, 0)

        drain(j0 + 1, 1)
        return carry

    lax.fori_loop(0, pairs, pair, 0)
    if nfull % 2:
        start(nfull - 1, 0)
        drain(nfull - 1, 0)
    if tail:
        toff = pl.multiple_of(nfull * _C, 8)
        pltpu.async_copy(
            table_sp.at[idx_v.at[pl.ds(toff, tail)]],
            rows1.at[pl.ds(0, tail)],
            g1,
        ).wait()
        pltpu.sync_copy(
            rows1.at[pl.ds(0, tail)], out_hbm.at[pl.ds(base + toff, tail)]
        )

  return _sc_gather_body


def _sc_gather(node_feat, idx_chunk, ec):
    per_w = ec // _NW
    mesh = plsc.VectorSubcoreMesh(core_axis_name="c", subcore_axis_name="s")
    fn = functools.partial(
        pl.kernel,
        mesh=mesh,
        out_type=jax.ShapeDtypeStruct((ec, _D), jnp.float32),
        scratch_types=[
            pltpu.VMEM_SHARED((_N, _D), jnp.float32),
            pltpu.VMEM((per_w,), jnp.int32),
            pltpu.VMEM((_C, _D), jnp.float32),
            pltpu.VMEM((_C, _D), jnp.float32),
            pltpu.SemaphoreType.DMA,
            pltpu.SemaphoreType.DMA,
        ],
    )(_make_sc_body(per_w))
    return fn(node_feat, idx_chunk)


_B = 6400
_G = _B // 128


def _tc_math(et, src, ef, emb, wur, wc, bu, br):
    iota_r = lax.broadcasted_iota(jnp.int32, (_RP, 128), 0)
    gs = []
    for t in range(_G):
        oht = (iota_r == et[0, t:t + 1, :]).astype(jnp.float32)
        gs.append(lax.dot_general(
            oht, emb, (((0,), (0,)), ((), ())),
            preferred_element_type=jnp.float32))
    g = jnp.concatenate(gs, axis=0)
    pur = jnp.dot(src, wur, preferred_element_type=jnp.float32)
    u = jax.nn.sigmoid(g[:, :_D] * ef + pur[:, :_D] + bu)
    r = jax.nn.sigmoid(g[:, _D:2 * _D] * ef + pur[:, _D:] + br)
    c = jnp.tanh(
        g[:, 2 * _D:] * ef
        + jnp.dot(r * src, wc, preferred_element_type=jnp.float32)
    )
    return u * c + (1.0 - u) * src


def _tc_body(et_ref, src_ref, ef_ref, emb_ref, wur_ref, wc_ref, bu_ref, br_ref,
             out_ref):
    out_ref[...] = _tc_math(
        et_ref[...], src_ref[...], ef_ref[...], emb_ref[...], wur_ref[...],
        wc_ref[...], bu_ref[...], br_ref[...])


def _tc_body_alias(et_ref, src_ref, ef_ref, emb_ref, wur_ref, wc_ref, bu_ref,
                   br_ref, buf_ref, out_ref):
    out_ref[...] = _tc_math(
        et_ref[...], src_ref[...], ef_ref[...], emb_ref[...], wur_ref[...],
        wc_ref[...], bu_ref[...], br_ref[...])


def _tc_chunk(et2, src_k, edge_feat, emb_all, wur, wc, bu, br, buf, row0, cb):
    base_specs = [
        pl.BlockSpec((1, _G, 128), lambda i, r=row0: (r + i, 0, 0)),
        pl.BlockSpec((_B, _D), lambda i: (i, 0)),
        pl.BlockSpec((_B, _D), lambda i, r=row0: (r + i, 0)),
        pl.BlockSpec((_RP, 3 * _D), lambda i: (0, 0)),
        pl.BlockSpec((_D, 2 * _D), lambda i: (0, 0)),
        pl.BlockSpec((_D, _D), lambda i: (0, 0)),
        pl.BlockSpec((1, _D), lambda i: (0, 0)),
        pl.BlockSpec((1, _D), lambda i: (0, 0)),
    ]
    out_spec = pl.BlockSpec((_B, _D), lambda i, r=row0: (r + i, 0))
    out_shape = jax.ShapeDtypeStruct((_E, _D), jnp.float32)
    if buf is None:
        return pl.pallas_call(
            _tc_body,
            grid=(cb,),
            in_specs=base_specs,
            out_specs=out_spec,
            out_shape=out_shape,
        )(et2, src_k, edge_feat, emb_all, wur, wc, bu, br)
    return pl.pallas_call(
        _tc_body_alias,
        grid=(cb,),
        in_specs=base_specs + [pl.BlockSpec((8, _D), lambda i: (0, 0))],
        out_specs=out_spec,
        out_shape=out_shape,
        input_output_aliases={8: 0},
    )(et2, src_k, edge_feat, emb_all, wur, wc, bu, br, buf)


def kernel(node_feat, edge_feat, src_idx, edge_type, emb_update, emb_reset,
           emb_candidate, W_update, b_update, W_reset, b_reset, W_candidate):
    src_idx = src_idx.astype(jnp.int32)
    edge_type = edge_type.astype(jnp.int32)
    emb_all = jnp.pad(
        jnp.concatenate([emb_update, emb_reset, emb_candidate], axis=1),
        ((0, _RP - _R), (0, 0)),
    )
    wur = jnp.concatenate([W_update.T, W_reset.T], axis=1)
    et3 = edge_type.reshape(_E // (_G * 128), _G, 128)
    bu = b_update.reshape(1, _D)
    br = b_reset.reshape(1, _D)
    wc = W_candidate.T

    src_chunks = [
        _sc_gather(node_feat,
                   lax.slice(src_idx, (_OFFS[k],), (_OFFS[k] + _CHUNKS[k],)),
                   _CHUNKS[k])
        for k in range(_K)
    ]
    buf = None
    for k in range(_K):
        buf = _tc_chunk(et3, src_chunks[k], edge_feat, emb_all, wur, wc, bu,
                        br, buf, _OFFS[k] // _B, _CHUNKS[k] // _B)
    return buf
